# baseline (speedup 1.0000x reference)
import functools

import jax
import jax.numpy as jnp
import numpy as np
from jax import lax
from jax.experimental import pallas as pl
from jax.experimental.pallas import tpu as pltpu
from jax.experimental.pallas import tpu_sc as plsc

NC = 2
NS = 16
NW = NC * NS


def _make_pool(B, L, E):
    RW = B // NW
    L8 = L // 8
    CB = 16
    NIDX = L * CB
    TCW = RW // 128
    GPT = 128 // CB
    NCHUNK = TCW * GPT
    NPAIR = NCHUNK // 2
    mesh = plsc.VectorSubcoreMesh(
        core_axis_name="c", subcore_axis_name="s",
        num_cores=NC, num_subcores=NS)

    @functools.partial(
        pl.kernel,
        mesh=mesh,
        compiler_params=pltpu.CompilerParams(
            use_tc_tiling_on_sc=False, needs_layout_passes=False),
        out_type=jax.ShapeDtypeStruct((B * E,), jnp.float32),
        scratch_types=[
            pltpu.VMEM((L8, 8, CB), jnp.int32),
            pltpu.VMEM((NIDX,), jnp.int32),
            pltpu.VMEM((NIDX,), jnp.int32),
            pltpu.VMEM((NIDX, E), jnp.bfloat16),
            pltpu.VMEM((NIDX, E), jnp.bfloat16),
            pltpu.VMEM((RW * E,), jnp.float32),
            pltpu.SemaphoreType.DMA,
            pltpu.SemaphoreType.DMA,
            pltpu.SemaphoreType.DMA,
        ],
    )
    def pool(xv_hbm, table_hbm, out_hbm,
             idx3, idx_a, idx_b, rows_a, rows_b, out_v, sem_a, sem_b, sem_i):
        wid = lax.axis_index("s") * NC + lax.axis_index("c")
        tc0 = wid * TCW

        def stage(g):
            tc = tc0 + g // GPT
            ln0 = (g % GPT) * CB
            pltpu.async_copy(xv_hbm.at[:, tc, :, pl.ds(ln0, CB)], idx3, sem_i)

        def drain_stage(g):
            tc = tc0 + g // GPT
            ln0 = (g % GPT) * CB
            pltpu.make_async_copy(xv_hbm.at[:, tc, :, pl.ds(ln0, CB)], idx3,
                                  sem_i).wait()

        def flatten(idx_f):
            def fbody(l1, carry):
                for l0 in range(8):
                    idx_f[pl.ds((l1 * 8 + l0) * CB, 16)] = idx3[l1, l0, :]
                return carry
            lax.fori_loop(0, L8, fbody, 0)

        MASK = jnp.int32(-65536)

        def reduce(rows_v, g):
            for b in range(CB):
                def rbody(l1, acc):
                    a0, a1, a2, a3 = acc
                    for l0 in range(8):
                        r = (l1 * 8 + l0) * CB + b
                        v = plsc.bitcast(rows_v[r, :], jnp.int32)
                        lo = plsc.bitcast(v << 16, jnp.float32)
                        hi = plsc.bitcast(v & MASK, jnp.float32)
                        if l0 % 2 == 0:
                            a0 = a0 + lo
                            a1 = a1 + hi
                        else:
                            a2 = a2 + lo
                            a3 = a3 + hi
                    return (a0, a1, a2, a3)

                z = jnp.zeros((16,), jnp.float32)
                a0, a1, a2, a3 = lax.fori_loop(0, L8, rbody, (z, z, z, z))
                ob = ((g // GPT) * 128 + (g % GPT) * CB + b) * E
                out_v[pl.ds(ob, 16)] = a0 + a2
                out_v[pl.ds(ob + 16, 16)] = a1 + a3

        stage(0)
        drain_stage(0)
        flatten(idx_a)
        pltpu.async_copy(table_hbm.at[idx_a], rows_a, sem_a)
        stage(1)
        drain_stage(1)
        flatten(idx_b)
        pltpu.async_copy(table_hbm.at[idx_b], rows_b, sem_b)

        def pair(p, carry):
            g0 = 2 * p
            not_last = p + 1 < NPAIR

            pltpu.make_async_copy(table_hbm.at[idx_a], rows_a, sem_a).wait()

            @pl.when(not_last)
            def _():
                stage(g0 + 2)

            reduce(rows_a, g0)

            @pl.when(not_last)
            def _():
                drain_stage(g0 + 2)
                flatten(idx_a)
                pltpu.async_copy(table_hbm.at[idx_a], rows_a, sem_a)

            pltpu.make_async_copy(table_hbm.at[idx_b], rows_b, sem_b).wait()

            @pl.when(not_last)
            def _():
                stage(g0 + 3)

            reduce(rows_b, g0 + 1)

            @pl.when(not_last)
            def _():
                drain_stage(g0 + 3)
                flatten(idx_b)
                pltpu.async_copy(table_hbm.at[idx_b], rows_b, sem_b)

            return carry

        lax.fori_loop(0, NPAIR, pair, 0)
        pltpu.sync_copy(out_v, out_hbm.at[pl.ds(tc0 * 128 * E, RW * E)])

    return pool


def _mlp_body(h_ref, w1_ref, b1_ref, w2_ref, b2_ref, o_ref):
    h = h_ref[...]
    z = jnp.dot(h, w1_ref[...], preferred_element_type=jnp.float32) + b1_ref[...]
    z = jnp.maximum(z, 0.0)
    o = jnp.dot(z, w2_ref[...], preferred_element_type=jnp.float32) + b2_ref[...]
    o_ref[...] = jax.nn.sigmoid(o)


def _make_mlp(B, E, H, O, BB):
    grid = (B // BB,)
    return pl.pallas_call(
        _mlp_body,
        grid=grid,
        in_specs=[
            pl.BlockSpec((BB, E), lambda i: (i, 0)),
            pl.BlockSpec((E, H), lambda i: (0, 0)),
            pl.BlockSpec((1, H), lambda i: (0, 0)),
            pl.BlockSpec((H, O), lambda i: (0, 0)),
            pl.BlockSpec((1, O), lambda i: (0, 0)),
        ],
        out_specs=pl.BlockSpec((BB, O), lambda i: (i, 0)),
        out_shape=jax.ShapeDtypeStruct((B, O), jnp.float32),
    )


def kernel(x, table, W1, b1, W2, b2):
    B, L = x.shape
    E = table.shape[1]
    H = W1.shape[1]
    O = W2.shape[1]
    xv = x.astype(jnp.int32).reshape(B // 128, 128, L // 8, 8).transpose(2, 0, 3, 1)
    pool = _make_pool(B, L, E)
    hsum = pool(xv, table.astype(jnp.bfloat16))
    h = hsum.reshape(B, E)
    perm = np.concatenate([np.arange(0, E, 2), np.arange(1, E, 2)])
    W1p = (W1 * (1.0 / L))[perm, :]
    mlp = _make_mlp(B, E, H, O, BB=2048)
    return mlp(h, W1p, b1.reshape(1, H), W2, b2.reshape(1, O))


# speedup vs baseline: 1.0979x; 1.0979x over previous
import functools

import jax
import jax.numpy as jnp
from jax import lax
from jax.experimental import pallas as pl
from jax.experimental.pallas import tpu as pltpu
from jax.experimental.pallas import tpu_sc as plsc

NC = 2
NS = 16
NW = NC * NS


def _make_pool(B, L, E):
    RW = B // NW
    L8 = L // 8
    CB = 16
    L1H = 12
    LH1 = L1H * 8
    N1 = LH1 * CB
    N2 = (L - LH1) * CB
    TCW = RW // 128
    GPT = 128 // CB
    NCHUNK = TCW * GPT
    mesh = plsc.VectorSubcoreMesh(
        core_axis_name="c", subcore_axis_name="s",
        num_cores=NC, num_subcores=NS)

    @functools.partial(
        pl.kernel,
        mesh=mesh,
        compiler_params=pltpu.CompilerParams(
            use_tc_tiling_on_sc=False, needs_layout_passes=False),
        out_type=jax.ShapeDtypeStruct((B * E,), jnp.float32),
        scratch_types=[
            pltpu.VMEM((L8, 8, CB), jnp.int32),
            pltpu.VMEM((N1,), jnp.int32),
            pltpu.VMEM((N1,), jnp.int32),
            pltpu.VMEM((N2,), jnp.int32),
            pltpu.VMEM((N2,), jnp.int32),
            pltpu.VMEM((N1, E), jnp.float32),
            pltpu.VMEM((N2, E), jnp.float32),
            pltpu.VMEM((RW * E,), jnp.float32),
            pltpu.SemaphoreType.DMA,
            pltpu.SemaphoreType.DMA,
            pltpu.SemaphoreType.DMA,
        ],
    )
    def pool(xv_hbm, table_hbm, out_hbm,
             idx3, if1a, if1b, if2a, if2b, rows1, rows2, out_v,
             sem_a, sem_b, sem_i):
        wid = lax.axis_index("s") * NC + lax.axis_index("c")
        tc0 = wid * TCW

        def stage(g):
            tc = tc0 + g // GPT
            ln0 = (g % GPT) * CB
            pltpu.async_copy(xv_hbm.at[:, tc, :, pl.ds(ln0, CB)], idx3, sem_i)

        def drain_stage(g):
            tc = tc0 + g // GPT
            ln0 = (g % GPT) * CB
            pltpu.make_async_copy(xv_hbm.at[:, tc, :, pl.ds(ln0, CB)], idx3,
                                  sem_i).wait()

        def flatten(if1, if2):
            def f1(l1, carry):
                for l0 in range(8):
                    if1[pl.ds((l1 * 8 + l0) * CB, 16)] = idx3[l1, l0, :]
                return carry
            lax.fori_loop(0, L1H, f1, 0)

            def f2(l1, carry):
                for l0 in range(8):
                    if2[pl.ds(((l1 - L1H) * 8 + l0) * CB, 16)] = idx3[l1, l0, :]
                return carry
            lax.fori_loop(L1H, L8, f2, 0)

        def g1_start(if1):
            pltpu.async_copy(table_hbm.at[if1], rows1, sem_a)

        def g1_wait(if1):
            pltpu.make_async_copy(table_hbm.at[if1], rows1, sem_a).wait()

        def g2_start(if2):
            pltpu.async_copy(table_hbm.at[if2], rows2, sem_b)

        def g2_wait(if2):
            pltpu.make_async_copy(table_hbm.at[if2], rows2, sem_b).wait()

        def reduce(rows_v, nl1, g, first):
            for b in range(CB):
                def rbody(l1, acc):
                    a0, a1, a2, a3 = acc
                    for l0 in range(8):
                        r = (l1 * 8 + l0) * CB + b
                        if l0 % 2 == 0:
                            a0 = a0 + rows_v[r, pl.ds(0, 16)]
                            a1 = a1 + rows_v[r, pl.ds(16, 16)]
                        else:
                            a2 = a2 + rows_v[r, pl.ds(0, 16)]
                            a3 = a3 + rows_v[r, pl.ds(16, 16)]
                    return (a0, a1, a2, a3)

                z = jnp.zeros((16,), jnp.float32)
                a0, a1, a2, a3 = lax.fori_loop(0, nl1, rbody, (z, z, z, z))
                ob = ((g // GPT) * 128 + (g % GPT) * CB + b) * E
                if first:
                    out_v[pl.ds(ob, 16)] = a0 + a2
                    out_v[pl.ds(ob + 16, 16)] = a1 + a3
                else:
                    out_v[pl.ds(ob, 16)] = out_v[pl.ds(ob, 16)] + (a0 + a2)
                    out_v[pl.ds(ob + 16, 16)] = (out_v[pl.ds(ob + 16, 16)]
                                                 + (a1 + a3))

        stage(0)
        drain_stage(0)
        flatten(if1a, if2a)
        g1_start(if1a)
        g2_start(if2a)

        def body(g, carry):
            not_last = g + 1 < NCHUNK
            even = g % 2 == 0

            @pl.when(not_last)
            def _():
                stage(g + 1)

            @pl.when(even)
            def _():
                g1_wait(if1a)
            @pl.when(jnp.logical_not(even))
            def _():
                g1_wait(if1b)

            reduce(rows1, L1H, g, True)

            @pl.when(not_last)
            def _():
                drain_stage(g + 1)

                @pl.when(even)
                def _():
                    flatten(if1b, if2b)
                    g1_start(if1b)
                @pl.when(jnp.logical_not(even))
                def _():
                    flatten(if1a, if2a)
                    g1_start(if1a)

            @pl.when(even)
            def _():
                g2_wait(if2a)
            @pl.when(jnp.logical_not(even))
            def _():
                g2_wait(if2b)

            reduce(rows2, L8 - L1H, g, False)

            @pl.when(not_last)
            def _():
                @pl.when(even)
                def _():
                    g2_start(if2b)
                @pl.when(jnp.logical_not(even))
                def _():
                    g2_start(if2a)

            return carry

        lax.fori_loop(0, NCHUNK, body, 0)
        pltpu.sync_copy(out_v, out_hbm.at[pl.ds(tc0 * 128 * E, RW * E)])

    return pool


def _mlp_body(h_ref, w1_ref, b1_ref, w2_ref, b2_ref, o_ref):
    h = h_ref[...]
    z = jnp.dot(h, w1_ref[...], preferred_element_type=jnp.float32) + b1_ref[...]
    z = jnp.maximum(z, 0.0)
    o = jnp.dot(z, w2_ref[...], preferred_element_type=jnp.float32) + b2_ref[...]
    o_ref[...] = jax.nn.sigmoid(o)


def _make_mlp(B, E, H, O, BB):
    grid = (B // BB,)
    return pl.pallas_call(
        _mlp_body,
        grid=grid,
        in_specs=[
            pl.BlockSpec((BB, E), lambda i: (i, 0)),
            pl.BlockSpec((E, H), lambda i: (0, 0)),
            pl.BlockSpec((1, H), lambda i: (0, 0)),
            pl.BlockSpec((H, O), lambda i: (0, 0)),
            pl.BlockSpec((1, O), lambda i: (0, 0)),
        ],
        out_specs=pl.BlockSpec((BB, O), lambda i: (i, 0)),
        out_shape=jax.ShapeDtypeStruct((B, O), jnp.float32),
    )


def kernel(x, table, W1, b1, W2, b2):
    B, L = x.shape
    E = table.shape[1]
    H = W1.shape[1]
    O = W2.shape[1]
    xv = x.astype(jnp.int32).reshape(B // 128, 128, L // 8, 8).transpose(2, 0, 3, 1)
    pool = _make_pool(B, L, E)
    hsum = pool(xv, table)
    h = hsum.reshape(B, E)
    mlp = _make_mlp(B, E, H, O, BB=2048)
    return mlp(h, W1 * (1.0 / L), b1.reshape(1, H), W2, b2.reshape(1, O))


# speedup vs baseline: 1.6886x; 1.5379x over previous
import functools

import jax
import jax.numpy as jnp
from jax import lax
from jax.experimental import pallas as pl
from jax.experimental.pallas import tpu as pltpu
from jax.experimental.pallas import tpu_sc as plsc

NC = 2
NS = 16
NW = NC * NS


def _make_pool(B, L, E):
    RW = B // NW
    L8 = L // 8
    CB = 16
    L1H = 12
    LH1 = L1H * 8
    N1 = LH1 * CB
    N2 = (L - LH1) * CB
    TCW = RW // 128
    GPT = 128 // CB
    NCHUNK = TCW * GPT
    mesh = plsc.VectorSubcoreMesh(
        core_axis_name="c", subcore_axis_name="s",
        num_cores=NC, num_subcores=NS)

    @functools.partial(
        pl.kernel,
        mesh=mesh,
        compiler_params=pltpu.CompilerParams(
            use_tc_tiling_on_sc=False, needs_layout_passes=False),
        out_type=jax.ShapeDtypeStruct((B * E,), jnp.float32),
        scratch_types=[
            pltpu.VMEM((L8, 8, CB), jnp.int32),
            pltpu.VMEM((N1,), jnp.int32),
            pltpu.VMEM((N1,), jnp.int32),
            pltpu.VMEM((N2,), jnp.int32),
            pltpu.VMEM((N2,), jnp.int32),
            pltpu.VMEM((N1, E), jnp.float32),
            pltpu.VMEM((N2, E), jnp.float32),
            pltpu.VMEM((RW * E,), jnp.float32),
            pltpu.SemaphoreType.DMA,
            pltpu.SemaphoreType.DMA,
            pltpu.SemaphoreType.DMA,
        ],
    )
    def pool(xv_hbm, table_hbm, out_hbm,
             idx3, if1a, if1b, if2a, if2b, rows1, rows2, out_v,
             sem_a, sem_b, sem_i):
        wid = lax.axis_index("s") * NC + lax.axis_index("c")
        tc0 = wid * TCW

        def stage(g):
            tc = tc0 + g // GPT
            ln0 = (g % GPT) * CB
            pltpu.async_copy(xv_hbm.at[:, tc, :, pl.ds(ln0, CB)], idx3, sem_i)

        def drain_stage(g):
            tc = tc0 + g // GPT
            ln0 = (g % GPT) * CB
            pltpu.make_async_copy(xv_hbm.at[:, tc, :, pl.ds(ln0, CB)], idx3,
                                  sem_i).wait()

        def phi(v):
            return ((v & jnp.int32(-8192))
                    | ((v & jnp.int32(2047)) << 2)
                    | ((v >> 11) & jnp.int32(3)))

        def flatten(if1, if2):
            def f1(l1, carry):
                for l0 in range(8):
                    if1[pl.ds((l1 * 8 + l0) * CB, 16)] = phi(idx3[l1, l0, :])
                return carry
            lax.fori_loop(0, L1H, f1, 0)

            def f2(l1, carry):
                for l0 in range(8):
                    if2[pl.ds(((l1 - L1H) * 8 + l0) * CB, 16)] = phi(
                        idx3[l1, l0, :])
                return carry
            lax.fori_loop(L1H, L8, f2, 0)

        def g1_start(if1):
            pltpu.async_copy(table_hbm.at[if1], rows1, sem_a)

        def g1_wait(if1):
            pltpu.make_async_copy(table_hbm.at[if1], rows1, sem_a).wait()

        def g2_start(if2):
            pltpu.async_copy(table_hbm.at[if2], rows2, sem_b)

        def g2_wait(if2):
            pltpu.make_async_copy(table_hbm.at[if2], rows2, sem_b).wait()

        def reduce(rows_v, nl1, g, first):
            for b in range(CB):
                def rbody(l1, acc):
                    a0, a1, a2, a3 = acc
                    for l0 in range(8):
                        r = (l1 * 8 + l0) * CB + b
                        if l0 % 2 == 0:
                            a0 = a0 + rows_v[r, pl.ds(0, 16)]
                            a1 = a1 + rows_v[r, pl.ds(16, 16)]
                        else:
                            a2 = a2 + rows_v[r, pl.ds(0, 16)]
                            a3 = a3 + rows_v[r, pl.ds(16, 16)]
                    return (a0, a1, a2, a3)

                z = jnp.zeros((16,), jnp.float32)
                a0, a1, a2, a3 = lax.fori_loop(0, nl1, rbody, (z, z, z, z))
                ob = ((g // GPT) * 128 + (g % GPT) * CB + b) * E
                if first:
                    out_v[pl.ds(ob, 16)] = a0 + a2
                    out_v[pl.ds(ob + 16, 16)] = a1 + a3
                else:
                    out_v[pl.ds(ob, 16)] = out_v[pl.ds(ob, 16)] + (a0 + a2)
                    out_v[pl.ds(ob + 16, 16)] = (out_v[pl.ds(ob + 16, 16)]
                                                 + (a1 + a3))

        stage(0)
        drain_stage(0)
        flatten(if1a, if2a)
        g1_start(if1a)
        g2_start(if2a)

        def body(g, carry):
            not_last = g + 1 < NCHUNK
            even = g % 2 == 0

            @pl.when(not_last)
            def _():
                stage(g + 1)

            @pl.when(even)
            def _():
                g1_wait(if1a)
            @pl.when(jnp.logical_not(even))
            def _():
                g1_wait(if1b)

            reduce(rows1, L1H, g, True)

            @pl.when(not_last)
            def _():
                drain_stage(g + 1)

                @pl.when(even)
                def _():
                    flatten(if1b, if2b)
                    g1_start(if1b)
                @pl.when(jnp.logical_not(even))
                def _():
                    flatten(if1a, if2a)
                    g1_start(if1a)

            @pl.when(even)
            def _():
                g2_wait(if2a)
            @pl.when(jnp.logical_not(even))
            def _():
                g2_wait(if2b)

            reduce(rows2, L8 - L1H, g, False)

            @pl.when(not_last)
            def _():
                @pl.when(even)
                def _():
                    g2_start(if2b)
                @pl.when(jnp.logical_not(even))
                def _():
                    g2_start(if2a)

            return carry

        lax.fori_loop(0, NCHUNK, body, 0)
        pltpu.sync_copy(out_v, out_hbm.at[pl.ds(tc0 * 128 * E, RW * E)])

    return pool


def _repack_body(t_ref, o_ref):
    x = t_ref[...]
    B4 = t_ref.shape[1] // 4
    cols = []
    for j in range(4):
        cols.append(jnp.swapaxes(x[:, j * B4:(j + 1) * B4], 0, 1))
    o_ref[...] = jnp.concatenate(cols, axis=1)


def _make_repack(V, E, BV):
    nblk = (V + BV - 1) // BV
    VP = nblk * BV
    return pl.pallas_call(
        _repack_body,
        grid=(nblk,),
        in_specs=[pl.BlockSpec((E, BV), lambda i: (0, i))],
        out_specs=pl.BlockSpec((BV // 4, 4 * E), lambda i: (i, 0)),
        out_shape=jax.ShapeDtypeStruct((VP // 4, 4 * E), jnp.float32),
    )


def _mlp_body(h_ref, w1_ref, b1_ref, w2_ref, b2_ref, o_ref):
    h = h_ref[...]
    z = jnp.dot(h, w1_ref[...], preferred_element_type=jnp.float32) + b1_ref[...]
    z = jnp.maximum(z, 0.0)
    o = jnp.dot(z, w2_ref[...], preferred_element_type=jnp.float32) + b2_ref[...]
    o_ref[...] = jax.nn.sigmoid(o)


def _make_mlp(B, E, H, O, BB):
    grid = (B // BB,)
    return pl.pallas_call(
        _mlp_body,
        grid=grid,
        in_specs=[
            pl.BlockSpec((BB, E), lambda i: (i, 0)),
            pl.BlockSpec((E, H), lambda i: (0, 0)),
            pl.BlockSpec((1, H), lambda i: (0, 0)),
            pl.BlockSpec((H, O), lambda i: (0, 0)),
            pl.BlockSpec((1, O), lambda i: (0, 0)),
        ],
        out_specs=pl.BlockSpec((BB, O), lambda i: (i, 0)),
        out_shape=jax.ShapeDtypeStruct((B, O), jnp.float32),
    )


def kernel(x, table, W1, b1, W2, b2):
    B, L = x.shape
    E = table.shape[1]
    H = W1.shape[1]
    O = W2.shape[1]
    xv = x.astype(jnp.int32).reshape(B // 128, 128, L // 8, 8).transpose(2, 0, 3, 1)
    V = table.shape[0]
    repack = _make_repack(V, E, BV=8192)
    tpack = repack(table.T)
    tlin = tpack.reshape(-1, E)
    pool = _make_pool(B, L, E)
    hsum = pool(xv, tlin)
    h = hsum.reshape(B, E)
    mlp = _make_mlp(B, E, H, O, BB=2048)
    return mlp(h, W1 * (1.0 / L), b1.reshape(1, H), W2, b2.reshape(1, O))
